# baseline (speedup 1.0000x reference)
import functools

import jax
import jax.numpy as jnp
from jax import lax
from jax.experimental import pallas as pl
from jax.experimental.pallas import tpu as pltpu
from jax.experimental.pallas import tpu_sc as plsc

_N_EDGES = 160000
_D = 256
_N_NODES = 10000
_HALF = 128
_LANES = 16
_IDXROWS = _N_EDGES // 128
_N_TILES = 16
_ROWS_PER = _IDXROWS // _N_TILES
_ROWS_REM = _IDXROWS - _ROWS_PER * _N_TILES
_NSLOTS = 3
_SG = 6
_NSG = _ROWS_PER // _SG
_NBLK = _N_NODES // 8
_NBLK_PER = _NBLK // _N_TILES
_NBLK_REM = _NBLK - _NBLK_PER * _N_TILES
_MAIN_ROWS = _NBLK_PER * 8
_CHUNK = 104
_NCHUNK = _MAIN_ROWS // _CHUNK
_SCALE = 0.25


def _sc_body(
    x_hbm, idx_hbm, out_hbm, acc,
    idx_sg, idx_tail, data_a, data_b, data_c,
    sem_a, sem_b, sem_c, sem_i, sem_z, sem_wa, sem_wb,
):
    c = lax.axis_index("c")
    s = lax.axis_index("s")
    col0 = c * _HALF

    node0 = (s * _NBLK_PER + jnp.minimum(s, _NBLK_REM)) * 8
    has_extra_nodes = s < _NBLK_REM
    extra_node0 = node0 + _MAIN_ROWS

    slots = [
        (data_a, sem_a),
        (data_b, sem_b),
        (data_c, sem_c),
    ]

    def _x_slice(row):
        return x_hbm.at[pl.ds(row * 128, 128), pl.ds(col0, _HALF)]

    base = s * _ROWS_PER + jnp.minimum(s, _ROWS_REM)
    has_extra_rows = s < _ROWS_REM
    cnt = _ROWS_PER + (s < _ROWS_REM).astype(jnp.int32)

    def _sg_slice(p):
        return idx_hbm.at[pl.ds(base + _SG * p, _SG)]

    tail_idx_src = idx_hbm.at[pl.ds(base + _ROWS_PER, 1)]

    pltpu.async_copy(_x_slice(base), data_a, sem_a)
    pltpu.async_copy(_x_slice(base + 1), data_b, sem_b)
    pltpu.async_copy(_sg_slice(0), idx_sg, sem_i)

    @pl.when(has_extra_rows)
    def _():
        pltpu.async_copy(tail_idx_src, idx_tail, sem_i)

    def _zero_row(i, carry):
        for q in range(_HALF // _LANES):
            data_c[i, pl.ds(q * _LANES, _LANES)] = jnp.zeros(
                (_LANES,), jnp.float32
            )
        return carry

    lax.fori_loop(0, _CHUNK, _zero_row, 0)
    for k in range(_NCHUNK):
        pltpu.async_copy(
            data_c.at[pl.ds(0, _CHUNK)],
            acc.at[pl.ds(node0 + k * _CHUNK, _CHUNK)],
            sem_z,
        )

    @pl.when(has_extra_nodes)
    def _():
        pltpu.async_copy(
            data_c.at[pl.ds(0, 8)], acc.at[pl.ds(extra_node0, 8)], sem_z
        )

    for k in range(_NCHUNK):
        pltpu.make_async_copy(
            data_c.at[pl.ds(0, _CHUNK)],
            acc.at[pl.ds(node0 + k * _CHUNK, _CHUNK)],
            sem_z,
        ).wait()

    @pl.when(has_extra_nodes)
    def _():
        pltpu.make_async_copy(
            data_c.at[pl.ds(0, 8)], acc.at[pl.ds(extra_node0, 8)], sem_z
        ).wait()

    pltpu.async_copy(_x_slice(base + 2), data_c, sem_c)
    plsc.subcore_barrier()

    def _sgroup(p, carry):
        t0 = base + _SG * p
        pltpu.make_async_copy(_sg_slice(p), idx_sg, sem_i).wait()
        for r in range(_SG):
            t = t0 + r
            data, sem = slots[r % _NSLOTS]
            pltpu.make_async_copy(_x_slice(t), data, sem).wait()
            pltpu.sync_copy(data, acc.at[idx_sg.at[r, 0]], add=True)

            @pl.when(t + _NSLOTS < base + cnt)
            def _():
                pltpu.async_copy(_x_slice(t + _NSLOTS), data, sem)

        @pl.when(p + 1 < _NSG)
        def _():
            pltpu.async_copy(_sg_slice(p + 1), idx_sg, sem_i)

        return carry

    lax.fori_loop(0, _NSG, _sgroup, 0)

    @pl.when(has_extra_rows)
    def _():
        b = base + _ROWS_PER
        pltpu.make_async_copy(tail_idx_src, idx_tail, sem_i).wait()
        pltpu.make_async_copy(_x_slice(b), data_a, sem_a).wait()
        pltpu.sync_copy(data_a, acc.at[idx_tail.at[0, 0]], add=True)

    plsc.subcore_barrier()

    bufs = [(data_a, sem_a), (data_b, sem_b)]

    def _acc_slice(k):
        return acc.at[pl.ds(node0 + k * _CHUNK, _CHUNK)]

    def _scale_rows(buf, n):
        def _scale_row(i, carry):
            for q in range(_HALF // _LANES):
                sl = pl.ds(q * _LANES, _LANES)
                buf[i, sl] = buf[i, sl] * _SCALE
            return carry

        lax.fori_loop(0, n, _scale_row, 0)

    wsems = [sem_wa, sem_wb]

    def _out_slice(k):
        return out_hbm.at[
            pl.ds(node0 + k * _CHUNK, _CHUNK), pl.ds(col0, _HALF)
        ]

    pltpu.async_copy(_acc_slice(0), data_a.at[pl.ds(0, _CHUNK)], sem_a)
    for k in range(_NCHUNK):
        buf, sem = bufs[k % 2]
        nbuf, nsem = bufs[(k + 1) % 2]
        wsem, nwsem = wsems[k % 2], wsems[(k + 1) % 2]
        pltpu.make_async_copy(_acc_slice(k), buf.at[pl.ds(0, _CHUNK)], sem).wait()
        if k + 1 < _NCHUNK:
            if k >= 1:
                pltpu.make_async_copy(
                    nbuf.at[pl.ds(0, _CHUNK)], _out_slice(k - 1), nwsem
                ).wait()
            pltpu.async_copy(
                _acc_slice(k + 1), nbuf.at[pl.ds(0, _CHUNK)], nsem
            )
        else:
            pltpu.make_async_copy(
                nbuf.at[pl.ds(0, _CHUNK)], _out_slice(k - 1), nwsem
            ).wait()

            @pl.when(has_extra_nodes)
            def _():
                pltpu.async_copy(
                    acc.at[pl.ds(extra_node0, 8)], nbuf.at[pl.ds(0, 8)], nsem
                )
        _scale_rows(buf, _CHUNK)
        pltpu.async_copy(buf.at[pl.ds(0, _CHUNK)], _out_slice(k), wsem)

    pltpu.make_async_copy(
        bufs[(_NCHUNK - 1) % 2][0].at[pl.ds(0, _CHUNK)],
        _out_slice(_NCHUNK - 1),
        wsems[(_NCHUNK - 1) % 2],
    ).wait()

    @pl.when(has_extra_nodes)
    def _():
        tbuf, tsem = bufs[_NCHUNK % 2]
        pltpu.make_async_copy(
            acc.at[pl.ds(extra_node0, 8)], tbuf.at[pl.ds(0, 8)], tsem
        ).wait()
        _scale_rows(tbuf, 8)
        pltpu.sync_copy(
            tbuf.at[pl.ds(0, 8)],
            out_hbm.at[pl.ds(extra_node0, 8), pl.ds(col0, _HALF)],
        )


@jax.jit
def _scatter_sc(x, idx1d):
    mesh = plsc.VectorSubcoreMesh(core_axis_name="c", subcore_axis_name="s")
    f = functools.partial(
        pl.kernel,
        out_type=jax.ShapeDtypeStruct((_N_NODES, _D), jnp.float32),
        mesh=mesh,
        scratch_types=[
            pltpu.VMEM_SHARED((_N_NODES, _HALF), jnp.float32),
            pltpu.VMEM((_SG, 1, 128), jnp.int32),
            pltpu.VMEM((1, 1, 128), jnp.int32),
            pltpu.VMEM((128, _HALF), jnp.float32),
            pltpu.VMEM((128, _HALF), jnp.float32),
            pltpu.VMEM((128, _HALF), jnp.float32),
            pltpu.SemaphoreType.DMA,
            pltpu.SemaphoreType.DMA,
            pltpu.SemaphoreType.DMA,
            pltpu.SemaphoreType.DMA,
            pltpu.SemaphoreType.DMA,
            pltpu.SemaphoreType.DMA,
            pltpu.SemaphoreType.DMA,
        ],
    )(_sc_body)
    return f(x, idx1d)


def kernel(x, index, dim, dim_size):
    idx = jnp.clip(
        index.astype(jnp.int32) + jnp.asarray(dim, jnp.int32),
        0,
        jnp.asarray(dim_size, jnp.int32) - 1,
    )
    return _scatter_sc(x, idx.reshape(_IDXROWS, 1, 128))


# speedup vs baseline: 1.0339x; 1.0339x over previous
import functools

import jax
import jax.numpy as jnp
from jax import lax
from jax.experimental import pallas as pl
from jax.experimental.pallas import tpu as pltpu
from jax.experimental.pallas import tpu_sc as plsc

_N_EDGES = 160000
_D = 256
_N_NODES = 10000
_HALF = 128
_LANES = 16
_N_TILES = 16
_BLK = 80
_NBLOCKS = _N_EDGES // _BLK
_BLK_PER = _NBLOCKS // _N_TILES
_NSLOTS = 4
_NBLK = _N_NODES // 8
_NBLK_PER = _NBLK // _N_TILES
_NBLK_REM = _NBLK - _NBLK_PER * _N_TILES
_MAIN_ROWS = _NBLK_PER * 8
_CHUNK = 48
_NCHUNK = _MAIN_ROWS // _CHUNK
_SCALE = 0.25


def _sc_body(
    x_hbm, idx_hbm, out_hbm, acc,
    idx_bufs, data_bufs, lsems, ssems, sem_z, sem_wa, sem_wb,
):
    c = lax.axis_index("c")
    s = lax.axis_index("s")
    col0 = c * _HALF

    node0 = (s * _NBLK_PER + jnp.minimum(s, _NBLK_REM)) * 8
    has_extra_nodes = s < _NBLK_REM
    extra_node0 = node0 + _MAIN_ROWS

    base = s * _BLK_PER

    def _x_slice(t):
        return x_hbm.at[pl.ds(t * _BLK, _BLK), pl.ds(col0, _HALF)]

    def _idx_slice(t):
        return idx_hbm.at[pl.ds(t * _BLK, _BLK)]

    def _load(t, r):
        pltpu.async_copy(_x_slice(t), data_bufs[r], lsems[r])
        pltpu.async_copy(_idx_slice(t), idx_bufs[r], lsems[r])

    def _load_wait(t, r):
        pltpu.make_async_copy(_x_slice(t), data_bufs[r], lsems[r]).wait()
        pltpu.make_async_copy(_idx_slice(t), idx_bufs[r], lsems[r]).wait()

    def _scat_start(r):
        pltpu.async_copy(data_bufs[r], acc.at[idx_bufs[r]], ssems[r], add=True)

    def _scat_wait(r):
        pltpu.make_async_copy(data_bufs[r], acc.at[idx_bufs[r]], ssems[r]).wait()

    for r in range(_NSLOTS - 1):
        _load(base + r, r)

    zstage = data_bufs[_NSLOTS - 1]

    def _zero_row(i, carry):
        for q in range(_HALF // _LANES):
            zstage[i, pl.ds(q * _LANES, _LANES)] = jnp.zeros(
                (_LANES,), jnp.float32
            )
        return carry

    lax.fori_loop(0, _CHUNK, _zero_row, 0)
    for k in range(_NCHUNK):
        pltpu.async_copy(
            zstage.at[pl.ds(0, _CHUNK)],
            acc.at[pl.ds(node0 + k * _CHUNK, _CHUNK)],
            sem_z,
        )

    @pl.when(has_extra_nodes)
    def _():
        pltpu.async_copy(
            zstage.at[pl.ds(0, 8)], acc.at[pl.ds(extra_node0, 8)], sem_z
        )

    for k in range(_NCHUNK):
        pltpu.make_async_copy(
            zstage.at[pl.ds(0, _CHUNK)],
            acc.at[pl.ds(node0 + k * _CHUNK, _CHUNK)],
            sem_z,
        ).wait()

    @pl.when(has_extra_nodes)
    def _():
        pltpu.make_async_copy(
            zstage.at[pl.ds(0, 8)], acc.at[pl.ds(extra_node0, 8)], sem_z
        ).wait()

    _load(base + _NSLOTS - 1, _NSLOTS - 1)
    plsc.subcore_barrier()

    def _quad(q, carry):
        j0 = 4 * q
        for r in range(_NSLOTS):
            j = j0 + r
            t = base + j
            _load_wait(t, r)
            _scat_start(r)

            @pl.when(j >= 1)
            def _():
                _scat_wait((r - 1) % _NSLOTS)

            @pl.when(jnp.logical_and(j >= 1, j + (_NSLOTS - 1) < _BLK_PER))
            def _():
                _load(t + _NSLOTS - 1, (r - 1) % _NSLOTS)

        return carry

    lax.fori_loop(0, _BLK_PER // _NSLOTS, _quad, 0)

    jt = _BLK_PER - 1
    _load_wait(base + jt, jt % _NSLOTS)
    _scat_start(jt % _NSLOTS)
    _scat_wait((jt - 1) % _NSLOTS)
    _scat_wait(jt % _NSLOTS)

    plsc.subcore_barrier()

    bufs = [(data_bufs[0], lsems[0]), (data_bufs[1], lsems[1])]
    wsems = [sem_wa, sem_wb]

    def _acc_slice(k):
        return acc.at[pl.ds(node0 + k * _CHUNK, _CHUNK)]

    def _out_slice(k):
        return out_hbm.at[
            pl.ds(node0 + k * _CHUNK, _CHUNK), pl.ds(col0, _HALF)
        ]

    def _scale_rows(buf, n):
        def _scale_row(i, carry):
            for q in range(_HALF // _LANES):
                sl = pl.ds(q * _LANES, _LANES)
                buf[i, sl] = buf[i, sl] * _SCALE
            return carry

        lax.fori_loop(0, n, _scale_row, 0)

    pltpu.async_copy(_acc_slice(0), data_bufs[0].at[pl.ds(0, _CHUNK)], lsems[0])
    for k in range(_NCHUNK):
        buf, sem = bufs[k % 2]
        nbuf, nsem = bufs[(k + 1) % 2]
        wsem, nwsem = wsems[k % 2], wsems[(k + 1) % 2]
        pltpu.make_async_copy(_acc_slice(k), buf.at[pl.ds(0, _CHUNK)], sem).wait()
        if k + 1 < _NCHUNK:
            if k >= 1:
                pltpu.make_async_copy(
                    nbuf.at[pl.ds(0, _CHUNK)], _out_slice(k - 1), nwsem
                ).wait()
            pltpu.async_copy(
                _acc_slice(k + 1), nbuf.at[pl.ds(0, _CHUNK)], nsem
            )
        else:
            pltpu.make_async_copy(
                nbuf.at[pl.ds(0, _CHUNK)], _out_slice(k - 1), nwsem
            ).wait()

            @pl.when(has_extra_nodes)
            def _():
                pltpu.async_copy(
                    acc.at[pl.ds(extra_node0, 8)], nbuf.at[pl.ds(0, 8)], nsem
                )
        _scale_rows(buf, _CHUNK)
        pltpu.async_copy(buf.at[pl.ds(0, _CHUNK)], _out_slice(k), wsem)

    pltpu.make_async_copy(
        bufs[(_NCHUNK - 1) % 2][0].at[pl.ds(0, _CHUNK)],
        _out_slice(_NCHUNK - 1),
        wsems[(_NCHUNK - 1) % 2],
    ).wait()

    @pl.when(has_extra_nodes)
    def _():
        tbuf, tsem = bufs[_NCHUNK % 2]
        pltpu.make_async_copy(
            acc.at[pl.ds(extra_node0, 8)], tbuf.at[pl.ds(0, 8)], tsem
        ).wait()
        _scale_rows(tbuf, 8)
        pltpu.sync_copy(
            tbuf.at[pl.ds(0, 8)],
            out_hbm.at[pl.ds(extra_node0, 8), pl.ds(col0, _HALF)],
        )


@jax.jit
def _scatter_sc(x, idx1d):
    mesh = plsc.VectorSubcoreMesh(core_axis_name="c", subcore_axis_name="s")

    def body(x_hbm, idx_hbm, out_hbm, acc, *rest):
        idx_bufs = list(rest[0:_NSLOTS])
        data_bufs = list(rest[_NSLOTS:2 * _NSLOTS])
        lsems = list(rest[2 * _NSLOTS:3 * _NSLOTS])
        ssems = list(rest[3 * _NSLOTS:4 * _NSLOTS])
        sem_z, sem_wa, sem_wb = rest[4 * _NSLOTS:]
        _sc_body(
            x_hbm, idx_hbm, out_hbm, acc,
            idx_bufs, data_bufs, lsems, ssems, sem_z, sem_wa, sem_wb,
        )

    f = functools.partial(
        pl.kernel,
        out_type=jax.ShapeDtypeStruct((_N_NODES, _D), jnp.float32),
        mesh=mesh,
        scratch_types=(
            [pltpu.VMEM_SHARED((_N_NODES, _HALF), jnp.float32)]
            + [pltpu.VMEM((_BLK,), jnp.int32) for _ in range(_NSLOTS)]
            + [pltpu.VMEM((_BLK, _HALF), jnp.float32) for _ in range(_NSLOTS)]
            + [pltpu.SemaphoreType.DMA for _ in range(2 * _NSLOTS + 3)]
        ),
    )(body)
    return f(x, idx1d)


def kernel(x, index, dim, dim_size):
    idx = jnp.clip(
        index.astype(jnp.int32) + jnp.asarray(dim, jnp.int32),
        0,
        jnp.asarray(dim_size, jnp.int32) - 1,
    )
    return _scatter_sc(x, idx)


# speedup vs baseline: 1.0416x; 1.0075x over previous
import functools

import jax
import jax.numpy as jnp
from jax import lax
from jax.experimental import pallas as pl
from jax.experimental.pallas import tpu as pltpu
from jax.experimental.pallas import tpu_sc as plsc

_N_EDGES = 160000
_D = 256
_N_NODES = 10000
_HALF = 128
_LANES = 16
_IDXROWS = _N_EDGES // 128
_N_TILES = 16
_ROWS_PER = _IDXROWS // _N_TILES
_ROWS_REM = _IDXROWS - _ROWS_PER * _N_TILES
_NSLOTS = 3
_NGROUPS = _ROWS_PER // _NSLOTS
_NBLK = _N_NODES // 8
_NBLK_PER = _NBLK // _N_TILES
_NBLK_REM = _NBLK - _NBLK_PER * _N_TILES
_MAIN_ROWS = _NBLK_PER * 8
_CHUNK = 104
_NCHUNK = _MAIN_ROWS // _CHUNK
_SCALE = 0.25


def _sc_body(
    x_hbm, idx_hbm, out_hbm, acc,
    idx_a, idx_b, idx_c, data_a, data_b, data_c,
    sem_a, sem_b, sem_c, sem_z, sem_wa, sem_wb,
):
    c = lax.axis_index("c")
    s = lax.axis_index("s")
    col0 = c * _HALF

    node0 = (s * _NBLK_PER + jnp.minimum(s, _NBLK_REM)) * 8
    has_extra_nodes = s < _NBLK_REM
    extra_node0 = node0 + _MAIN_ROWS

    slots = [
        (data_a, idx_a, sem_a),
        (data_b, idx_b, sem_b),
        (data_c, idx_c, sem_c),
    ]

    def _x_slice(row):
        return x_hbm.at[pl.ds(row * 128, 128), pl.ds(col0, _HALF)]

    def _idx_slice(row):
        return idx_hbm.at[pl.ds(row * 128, 128)]

    def _issue(row, data, idxb, sem):
        pltpu.async_copy(_x_slice(row), data, sem)
        pltpu.async_copy(_idx_slice(row), idxb, sem)

    def _wait(row, data, idxb, sem):
        pltpu.make_async_copy(_x_slice(row), data, sem).wait()
        pltpu.make_async_copy(_idx_slice(row), idxb, sem).wait()

    base = s * _ROWS_PER + jnp.minimum(s, _ROWS_REM)
    has_extra_rows = s < _ROWS_REM
    cnt = _ROWS_PER + (s < _ROWS_REM).astype(jnp.int32)

    _issue(base, data_a, idx_a, sem_a)
    _issue(base + 1, data_b, idx_b, sem_b)

    def _zero_row(i, carry):
        for q in range(_HALF // _LANES):
            data_c[i, pl.ds(q * _LANES, _LANES)] = jnp.zeros(
                (_LANES,), jnp.float32
            )
        return carry

    lax.fori_loop(0, _CHUNK, _zero_row, 0)
    for k in range(_NCHUNK):
        pltpu.async_copy(
            data_c.at[pl.ds(0, _CHUNK)],
            acc.at[pl.ds(node0 + k * _CHUNK, _CHUNK)],
            sem_z,
        )

    @pl.when(has_extra_nodes)
    def _():
        pltpu.async_copy(
            data_c.at[pl.ds(0, 8)], acc.at[pl.ds(extra_node0, 8)], sem_z
        )

    for k in range(_NCHUNK):
        pltpu.make_async_copy(
            data_c.at[pl.ds(0, _CHUNK)],
            acc.at[pl.ds(node0 + k * _CHUNK, _CHUNK)],
            sem_z,
        ).wait()

    @pl.when(has_extra_nodes)
    def _():
        pltpu.make_async_copy(
            data_c.at[pl.ds(0, 8)], acc.at[pl.ds(extra_node0, 8)], sem_z
        ).wait()

    _issue(base + 2, data_c, idx_c, sem_c)
    plsc.subcore_barrier()

    def _group(j, carry):
        t0 = base + _NSLOTS * j
        for r, (data, idxb, sem) in enumerate(slots):
            t = t0 + r
            _wait(t, data, idxb, sem)
            pltpu.sync_copy(data, acc.at[idxb], add=True)

            @pl.when(t + _NSLOTS < base + cnt)
            def _():
                _issue(t + _NSLOTS, data, idxb, sem)

        return carry

    lax.fori_loop(0, _NGROUPS, _group, 0)

    @pl.when(has_extra_rows)
    def _():
        b = base + _ROWS_PER
        _wait(b, data_a, idx_a, sem_a)
        pltpu.sync_copy(data_a, acc.at[idx_a], add=True)

    plsc.subcore_barrier()

    bufs = [(data_a, sem_a), (data_b, sem_b)]

    def _acc_slice(k):
        return acc.at[pl.ds(node0 + k * _CHUNK, _CHUNK)]

    def _scale_rows(buf, n):
        def _scale_row(i, carry):
            for q in range(_HALF // _LANES):
                sl = pl.ds(q * _LANES, _LANES)
                buf[i, sl] = buf[i, sl] * _SCALE
            return carry

        lax.fori_loop(0, n, _scale_row, 0)

    wsems = [sem_wa, sem_wb]

    def _out_slice(k):
        return out_hbm.at[
            pl.ds(node0 + k * _CHUNK, _CHUNK), pl.ds(col0, _HALF)
        ]

    pltpu.async_copy(_acc_slice(0), data_a.at[pl.ds(0, _CHUNK)], sem_a)
    for k in range(_NCHUNK):
        buf, sem = bufs[k % 2]
        nbuf, nsem = bufs[(k + 1) % 2]
        wsem, nwsem = wsems[k % 2], wsems[(k + 1) % 2]
        pltpu.make_async_copy(_acc_slice(k), buf.at[pl.ds(0, _CHUNK)], sem).wait()
        if k + 1 < _NCHUNK:
            if k >= 1:
                pltpu.make_async_copy(
                    nbuf.at[pl.ds(0, _CHUNK)], _out_slice(k - 1), nwsem
                ).wait()
            pltpu.async_copy(
                _acc_slice(k + 1), nbuf.at[pl.ds(0, _CHUNK)], nsem
            )
        else:
            pltpu.make_async_copy(
                nbuf.at[pl.ds(0, _CHUNK)], _out_slice(k - 1), nwsem
            ).wait()

            @pl.when(has_extra_nodes)
            def _():
                pltpu.async_copy(
                    acc.at[pl.ds(extra_node0, 8)], nbuf.at[pl.ds(0, 8)], nsem
                )
        _scale_rows(buf, _CHUNK)
        pltpu.async_copy(buf.at[pl.ds(0, _CHUNK)], _out_slice(k), wsem)

    pltpu.make_async_copy(
        bufs[(_NCHUNK - 1) % 2][0].at[pl.ds(0, _CHUNK)],
        _out_slice(_NCHUNK - 1),
        wsems[(_NCHUNK - 1) % 2],
    ).wait()

    @pl.when(has_extra_nodes)
    def _():
        tbuf, tsem = bufs[_NCHUNK % 2]
        pltpu.make_async_copy(
            acc.at[pl.ds(extra_node0, 8)], tbuf.at[pl.ds(0, 8)], tsem
        ).wait()
        _scale_rows(tbuf, 8)
        pltpu.sync_copy(
            tbuf.at[pl.ds(0, 8)],
            out_hbm.at[pl.ds(extra_node0, 8), pl.ds(col0, _HALF)],
        )


@jax.jit
def _scatter_sc(x, idx1d):
    mesh = plsc.VectorSubcoreMesh(core_axis_name="c", subcore_axis_name="s")
    f = functools.partial(
        pl.kernel,
        out_type=jax.ShapeDtypeStruct((_N_NODES, _D), jnp.float32),
        mesh=mesh,
        scratch_types=[
            pltpu.VMEM_SHARED((_N_NODES, _HALF), jnp.float32),
            pltpu.VMEM((128,), jnp.int32),
            pltpu.VMEM((128,), jnp.int32),
            pltpu.VMEM((128,), jnp.int32),
            pltpu.VMEM((128, _HALF), jnp.float32),
            pltpu.VMEM((128, _HALF), jnp.float32),
            pltpu.VMEM((128, _HALF), jnp.float32),
            pltpu.SemaphoreType.DMA,
            pltpu.SemaphoreType.DMA,
            pltpu.SemaphoreType.DMA,
            pltpu.SemaphoreType.DMA,
            pltpu.SemaphoreType.DMA,
            pltpu.SemaphoreType.DMA,
        ],
    )(_sc_body)
    return f(x, idx1d)


def kernel(x, index, dim, dim_size):
    idx = jnp.clip(
        index.astype(jnp.int32) + jnp.asarray(dim, jnp.int32),
        0,
        jnp.asarray(dim_size, jnp.int32) - 1,
    )
    return _scatter_sc(x, idx)
